# speedup vs baseline: 2.3536x; 2.3536x over previous
import functools

import jax
import jax.numpy as jnp
from jax import lax
from jax.experimental import pallas as pl
from jax.experimental.pallas import tpu as pltpu
from jax.experimental.pallas import tpu_sc as plsc

D_MODEL = 512
CHUNK = 64


@functools.lru_cache(maxsize=None)
def _build_lookup(total, d_model):
    info = plsc.get_sparse_core_info()
    num_cores, num_subcores = info.num_cores, info.num_subcores
    nw = num_cores * num_subcores
    assert total % (nw * CHUNK) == 0
    b_per_w = total // nw
    n_chunks = b_per_w // CHUNK
    assert n_chunks % 2 == 0
    n_pairs = n_chunks // 2

    mesh = plsc.VectorSubcoreMesh(core_axis_name="c", subcore_axis_name="s")

    @functools.partial(
        pl.kernel,
        mesh=mesh,
        out_type=jax.ShapeDtypeStruct((total, d_model), jnp.float32),
        scratch_types=[
            pltpu.VMEM((n_chunks, CHUNK), jnp.int32),
            pltpu.VMEM((2, CHUNK, d_model), jnp.float32),
            pltpu.SemaphoreType.DMA,
            pltpu.SemaphoreType.DMA,
        ],
    )
    def lookup(idx_hbm, table_hbm, out_hbm, idx_v, rows_v, sem0, sem1):
        wid = lax.axis_index("s") * num_cores + lax.axis_index("c")
        base = wid * b_per_w
        pltpu.sync_copy(idx_hbm.at[wid], idx_v)

        pltpu.make_async_copy(
            table_hbm.at[idx_v.at[0]], rows_v.at[0], sem0
        ).start()

        def body(i, carry):
            c0 = 2 * i
            pltpu.make_async_copy(
                table_hbm.at[idx_v.at[c0 + 1]], rows_v.at[1], sem1
            ).start()
            pltpu.make_async_copy(
                table_hbm.at[idx_v.at[c0]], rows_v.at[0], sem0
            ).wait()
            pltpu.sync_copy(
                rows_v.at[0], out_hbm.at[pl.ds(base + c0 * CHUNK, CHUNK)]
            )

            @pl.when(i + 1 < n_pairs)
            def _():
                pltpu.make_async_copy(
                    table_hbm.at[idx_v.at[c0 + 2]], rows_v.at[0], sem0
                ).start()

            pltpu.make_async_copy(
                table_hbm.at[idx_v.at[c0 + 1]], rows_v.at[1], sem1
            ).wait()
            pltpu.sync_copy(
                rows_v.at[1], out_hbm.at[pl.ds(base + (c0 + 1) * CHUNK, CHUNK)]
            )
            return carry

        lax.fori_loop(0, n_pairs, body, 0)

    return lookup, nw, n_chunks


def kernel(x, weight):
    batch, hist = x.shape
    total = batch * hist
    d_model = weight.shape[1]
    lookup, nw, n_chunks = _build_lookup(total, d_model)
    idx = x.reshape(nw, n_chunks, CHUNK).astype(jnp.int32)
    out = lookup(idx, weight)
    return out.reshape(batch, hist, d_model)


# speedup vs baseline: 2.3595x; 1.0025x over previous
import functools

import jax
import jax.numpy as jnp
from jax import lax
from jax.experimental import pallas as pl
from jax.experimental.pallas import tpu as pltpu
from jax.experimental.pallas import tpu_sc as plsc

D_MODEL = 512
CHUNK = 80


@functools.lru_cache(maxsize=None)
def _build_lookup(total, d_model):
    info = plsc.get_sparse_core_info()
    num_cores, num_subcores = info.num_cores, info.num_subcores
    nw = num_cores * num_subcores
    assert total % (nw * CHUNK) == 0
    b_per_w = total // nw
    n_chunks = b_per_w // CHUNK
    assert n_chunks % 2 == 0
    n_pairs = n_chunks // 2

    mesh = plsc.VectorSubcoreMesh(core_axis_name="c", subcore_axis_name="s")

    @functools.partial(
        pl.kernel,
        mesh=mesh,
        out_type=jax.ShapeDtypeStruct((total, d_model), jnp.float32),
        scratch_types=[
            pltpu.VMEM((n_chunks, CHUNK), jnp.int32),
            pltpu.VMEM((2, CHUNK, d_model), jnp.float32),
            pltpu.SemaphoreType.DMA,
            pltpu.SemaphoreType.DMA,
        ],
    )
    def lookup(idx_hbm, table_hbm, out_hbm, idx_v, rows_v, sem0, sem1):
        wid = lax.axis_index("s") * num_cores + lax.axis_index("c")
        base = wid * b_per_w
        pltpu.sync_copy(idx_hbm.at[wid], idx_v)

        pltpu.make_async_copy(
            table_hbm.at[idx_v.at[0]], rows_v.at[0], sem0
        ).start()

        def body(i, carry):
            c0 = 2 * i
            pltpu.make_async_copy(
                table_hbm.at[idx_v.at[c0 + 1]], rows_v.at[1], sem1
            ).start()
            pltpu.make_async_copy(
                table_hbm.at[idx_v.at[c0]], rows_v.at[0], sem0
            ).wait()
            pltpu.sync_copy(
                rows_v.at[0], out_hbm.at[pl.ds(base + c0 * CHUNK, CHUNK)]
            )

            @pl.when(i + 1 < n_pairs)
            def _():
                pltpu.make_async_copy(
                    table_hbm.at[idx_v.at[c0 + 2]], rows_v.at[0], sem0
                ).start()

            pltpu.make_async_copy(
                table_hbm.at[idx_v.at[c0 + 1]], rows_v.at[1], sem1
            ).wait()
            pltpu.sync_copy(
                rows_v.at[1], out_hbm.at[pl.ds(base + (c0 + 1) * CHUNK, CHUNK)]
            )
            return carry

        lax.fori_loop(0, n_pairs, body, 0)

    return lookup, nw, n_chunks


def kernel(x, weight):
    batch, hist = x.shape
    total = batch * hist
    d_model = weight.shape[1]
    lookup, nw, n_chunks = _build_lookup(total, d_model)
    idx = x.reshape(nw, n_chunks, CHUNK).astype(jnp.int32)
    out = lookup(idx, weight)
    return out.reshape(batch, hist, d_model)
